# baseline (speedup 1.0000x reference)
import math

import jax
import jax.numpy as jnp
from jax import lax
from jax.experimental import pallas as pl
from jax.experimental.pallas import tpu as pltpu
from jax.experimental.pallas import tpu_sc as plsc

N = 10000
E = 320000
D0 = 128
DP = 16
NP = 10240
NCORE = 2
NSUB = 16
NW = NCORE * NSUB
EPW = 10080
EP = EPW * NW
KB = 3
CH = 56
NGRP = EPW // (CH * KB)
CH1 = 112
NGRP1 = EPW // (CH1 * KB)
RPT = NP // NSUB
_BNI = 1.0 / math.sqrt(1.0 + 1e-5)

_sc_mesh = plsc.VectorSubcoreMesh(
    core_axis_name="c", subcore_axis_name="s", num_cores=NCORE,
    num_subcores=NSUB)


def _ring(tbl, src_hbm, dst_hbm, acc, src_v, dst_v, rows_v, isem, gsem, ssem,
          ebase, ngrp, ch, ones_v=None, accd=None, osem=None):
    def fire_idx(g, poff):
        for b in range(KB):
            off = ebase + (g * KB + b) * ch
            pltpu.async_copy(src_hbm.at[pl.ds(off, ch)], src_v.at[poff + b],
                             isem)
            pltpu.async_copy(dst_hbm.at[pl.ds(off, ch)], dst_v.at[poff + b],
                             isem)

    def drain_idx(poff):
        for b in range(KB):
            pltpu.make_async_copy(src_hbm.at[pl.ds(ebase, ch)],
                                  src_v.at[poff + b], isem).wait()
            pltpu.make_async_copy(dst_hbm.at[pl.ds(ebase, ch)],
                                  dst_v.at[poff + b], isem).wait()

    def drain_scatter(poff):
        for b in range(KB):
            pltpu.make_async_copy(rows_v.at[poff + b],
                                  acc.at[dst_v.at[poff + b]], ssem).wait()
            if accd is not None:
                pltpu.make_async_copy(ones_v,
                                      accd.at[dst_v.at[poff + b]], osem).wait()

    fire_idx(0, 0)

    def group(g, carry):
        poff = lax.rem(g, 2) * KB
        pnoff = KB - poff
        drain_idx(poff)
        gc = []
        for b in range(KB):
            gc.append(pltpu.async_copy(tbl.at[src_v.at[poff + b]],
                                       rows_v.at[poff + b], gsem[b]))

        @pl.when(g > 0)
        def _():
            drain_scatter(pnoff)

        fire_idx(jnp.minimum(g + 1, ngrp - 1), pnoff)
        for b in range(KB):
            gc[b].wait()
            pltpu.async_copy(rows_v.at[poff + b], acc.at[dst_v.at[poff + b]],
                             ssem, add=True)
            if accd is not None:
                pltpu.async_copy(ones_v, accd.at[dst_v.at[poff + b]], osem,
                                 add=True)
        return carry

    lax.fori_loop(0, ngrp, group, 0)
    last_poff = ((ngrp - 1) % 2) * KB
    drain_scatter(last_poff)
    drain_idx(KB - last_poff)


def _agg_body(h_hbm, src_hbm, dst_hbm, zrow_hbm, zdeg_hbm,
              p_out, deg_out, acc, accd, src_v, dst_v, ones_v, rows_v,
              isem, gsem, ssem, osem):
    cid = lax.axis_index("c")
    sid = lax.axis_index("s")
    wid = cid * NSUB + sid

    pltpu.sync_copy(zrow_hbm, acc.at[pl.ds(sid * RPT, RPT)])
    pltpu.sync_copy(zdeg_hbm, accd.at[pl.ds(sid * RPT, RPT)])
    ones_v[...] = jnp.ones((CH,), jnp.float32)
    plsc.subcore_barrier()

    _ring(h_hbm, src_hbm, dst_hbm, acc, src_v, dst_v, rows_v, isem, gsem,
          ssem, wid * EPW, NGRP, CH, ones_v=ones_v, accd=accd, osem=osem)
    plsc.subcore_barrier()

    pltpu.sync_copy(acc.at[pl.ds(sid * RPT, RPT)],
                    p_out.at[cid, pl.ds(sid * RPT, RPT)])
    pltpu.sync_copy(accd.at[pl.ds(sid * RPT, RPT)],
                    deg_out.at[cid, pl.ds(sid * RPT, RPT)])


def _agg128(h, src, dst, zrow, zdeg):
    return pl.kernel(
        _agg_body,
        out_type=(jax.ShapeDtypeStruct((NCORE, NP, D0), jnp.float32),
                  jax.ShapeDtypeStruct((NCORE, NP), jnp.float32)),
        mesh=_sc_mesh,
        scratch_types=[
            pltpu.VMEM_SHARED((NP, D0), jnp.float32),
            pltpu.VMEM_SHARED((NP,), jnp.float32),
            pltpu.VMEM((2 * KB, CH), jnp.int32),
            pltpu.VMEM((2 * KB, CH), jnp.int32),
            pltpu.VMEM((CH,), jnp.float32),
            pltpu.VMEM((2 * KB, CH, D0), jnp.float32),
            pltpu.SemaphoreType.DMA,
            [pltpu.SemaphoreType.DMA] * KB,
            pltpu.SemaphoreType.DMA,
            pltpu.SemaphoreType.DMA,
        ],
    )(h, src, dst, zrow, zdeg)


def _agg16_body(z_hbm, src_hbm, dst_hbm, z16_hbm, q_out,
                acc, src_v, dst_v, rows_v, isem, gsem, ssem):
    cid = lax.axis_index("c")
    sid = lax.axis_index("s")
    wid = cid * NSUB + sid

    pltpu.sync_copy(z16_hbm, acc.at[pl.ds(sid * RPT, RPT)])
    plsc.subcore_barrier()

    _ring(z_hbm, src_hbm, dst_hbm, acc, src_v, dst_v, rows_v, isem, gsem,
          ssem, wid * EPW, NGRP1, CH1)
    plsc.subcore_barrier()

    pltpu.sync_copy(acc.at[pl.ds(sid * RPT, RPT)],
                    q_out.at[cid, pl.ds(sid * RPT, RPT)])


def _agg16(z1, src, dst, z16):
    return pl.kernel(
        _agg16_body,
        out_type=jax.ShapeDtypeStruct((NCORE, NP, DP), jnp.float32),
        mesh=_sc_mesh,
        compiler_params=pltpu.CompilerParams(use_tc_tiling_on_sc=False),
        scratch_types=[
            pltpu.VMEM_SHARED((NP, DP), jnp.float32),
            pltpu.VMEM((2 * KB, CH1), jnp.int32),
            pltpu.VMEM((2 * KB, CH1), jnp.int32),
            pltpu.VMEM((2 * KB, CH1, DP), jnp.float32),
            pltpu.SemaphoreType.DMA,
            [pltpu.SemaphoreType.DMA] * KB,
            pltpu.SemaphoreType.DMA,
        ],
    )(z1, src, dst, z16)


def _mid_body(eps_ref, h_r, p0_r, p1_r, deg_r, w0_r, b0_r, g0_r, be0_r,
              w1_r, wp_r, z1_o, zp_o, dinv_o):
    deg = deg_r[0, :] + deg_r[1, :]
    dinv = 1.0 / jnp.maximum(deg, 1.0)
    neigh = (p0_r[...] + p1_r[...]) * dinv[:, None]
    t0 = (1.0 + eps_ref[0]) * h_r[...] + neigh
    a = jnp.dot(t0, w0_r[...], preferred_element_type=jnp.float32) + b0_r[...]
    h0 = jnp.maximum(a * (g0_r[...] * _BNI) + be0_r[...], 0.0)
    z1_o[...] = jnp.dot(h0, w1_r[...], preferred_element_type=jnp.float32)
    zp_o[...] = jnp.dot(h0, wp_r[...], preferred_element_type=jnp.float32)
    dinv_o[...] = jnp.broadcast_to(dinv[:, None], dinv_o.shape)


def _mid(hp, p, degp, eps0, W0, b0, g0, be0, W1p, Wpp):
    R = 1024
    grid = NP // R
    row = lambda i: (i, 0)
    full = lambda i: (0, 0)
    return pl.pallas_call(
        _mid_body,
        grid=(grid,),
        in_specs=[
            pl.BlockSpec(memory_space=pltpu.SMEM),
            pl.BlockSpec((R, D0), row),
            pl.BlockSpec((R, D0), row),
            pl.BlockSpec((R, D0), row),
            pl.BlockSpec((NCORE, R), lambda i: (0, i)),
            pl.BlockSpec((D0, D0), full),
            pl.BlockSpec((1, D0), full),
            pl.BlockSpec((1, D0), full),
            pl.BlockSpec((1, D0), full),
            pl.BlockSpec((D0, DP), full),
            pl.BlockSpec((D0, DP), full),
        ],
        out_specs=[
            pl.BlockSpec((R, DP), row),
            pl.BlockSpec((R, DP), row),
            pl.BlockSpec((R, DP), row),
        ],
        out_shape=[
            jax.ShapeDtypeStruct((NP, DP), jnp.float32),
            jax.ShapeDtypeStruct((NP, DP), jnp.float32),
            jax.ShapeDtypeStruct((NP, DP), jnp.float32),
        ],
    )(eps0, hp, p[0], p[1], degp, W0, b0, g0, be0, W1p, Wpp)


def _fin_body(eps_ref, z1_r, q0_r, q1_r, dinv_r, zp_r, b1_r, g1_r, be1_r, o_r):
    t1 = (1.0 + eps_ref[0]) * z1_r[...] + (q0_r[...] + q1_r[...]) * dinv_r[...]
    h1 = jnp.maximum((t1 + b1_r[...]) * (g1_r[...] * _BNI) + be1_r[...], 0.0)
    o_r[...] = (zp_r[...] + h1) * 0.5


def _fin(z1, q, dinv, zp, eps1, b1p, g1p, be1p):
    R = 1024
    grid = NP // R
    row = lambda i: (i, 0)
    full = lambda i: (0, 0)
    return pl.pallas_call(
        _fin_body,
        grid=(grid,),
        in_specs=[
            pl.BlockSpec(memory_space=pltpu.SMEM),
            pl.BlockSpec((R, DP), row),
            pl.BlockSpec((R, DP), row),
            pl.BlockSpec((R, DP), row),
            pl.BlockSpec((R, DP), row),
            pl.BlockSpec((R, DP), row),
            pl.BlockSpec((1, DP), full),
            pl.BlockSpec((1, DP), full),
            pl.BlockSpec((1, DP), full),
        ],
        out_specs=pl.BlockSpec((R, DP), row),
        out_shape=jax.ShapeDtypeStruct((NP, DP), jnp.float32),
    )(eps1, z1, q[0], q[1], dinv, zp, b1p, g1p, be1p)


def kernel(g, h, snorm_n, snorm_e, eps0, W0, b0, bn0_g, bn0_b,
           eps1, W1, b1, bn1_g, bn1_b, Wp):
    del snorm_n, snorm_e
    f32 = jnp.float32
    npad = EP - E
    fill = jnp.arange(npad, dtype=jnp.int32)
    src = jnp.concatenate([g[0], fill % 997])
    dst = jnp.concatenate([g[1], N + fill % (NP - N)])
    hp = jnp.pad(h, ((0, NP - N), (0, 0)))
    zrow = jnp.zeros((RPT, D0), f32)
    zdeg = jnp.zeros((RPT,), f32)
    z16 = jnp.zeros((RPT, DP), f32)
    W1p = jnp.pad(W1, ((0, 0), (0, DP - W1.shape[1])))
    Wpp = jnp.pad(Wp, ((0, 0), (0, DP - Wp.shape[1])))
    b1p = jnp.pad(b1, (0, DP - b1.shape[0])).reshape(1, DP)
    g1p = jnp.pad(bn1_g, (0, DP - bn1_g.shape[0])).reshape(1, DP)
    be1p = jnp.pad(bn1_b, (0, DP - bn1_b.shape[0])).reshape(1, DP)

    p, degp = _agg128(h, src, dst, zrow, zdeg)
    z1, zp, dinv = _mid(hp, p, degp, eps0, W0, b0.reshape(1, D0),
                        bn0_g.reshape(1, D0), bn0_b.reshape(1, D0), W1p, Wpp)
    q = _agg16(z1, src, dst, z16)
    out = _fin(z1, q, dinv, zp, eps1, b1p, g1p, be1p)
    return out[:N, :Wp.shape[1]]


# speedup vs baseline: 1.0295x; 1.0295x over previous
import math

import jax
import jax.numpy as jnp
from jax import lax
from jax.experimental import pallas as pl
from jax.experimental.pallas import tpu as pltpu
from jax.experimental.pallas import tpu_sc as plsc

N = 10000
E = 320000
D0 = 128
DP = 16
NP = 10240
NCORE = 2
NSUB = 16
NW = NCORE * NSUB
EPW = 10080
EP = EPW * NW
KB = 7
CH = 24
NGRP = EPW // (CH * KB)
CH1 = 96
NGRP1 = EPW // (CH1 * KB)
RPT = NP // NSUB
_BNI = 1.0 / math.sqrt(1.0 + 1e-5)

_sc_mesh = plsc.VectorSubcoreMesh(
    core_axis_name="c", subcore_axis_name="s", num_cores=NCORE,
    num_subcores=NSUB)


def _ring(tbl, src_hbm, dst_hbm, acc, src_v, dst_v, rows_v, isem, gsem, ssem,
          ebase, ngrp, ch, ones_v=None, accd=None, osem=None):
    def fire_idx(g, poff):
        for b in range(KB):
            off = ebase + (g * KB + b) * ch
            pltpu.async_copy(src_hbm.at[pl.ds(off, ch)], src_v.at[poff + b],
                             isem)
            pltpu.async_copy(dst_hbm.at[pl.ds(off, ch)], dst_v.at[poff + b],
                             isem)

    def drain_idx(poff):
        for b in range(KB):
            pltpu.make_async_copy(src_hbm.at[pl.ds(ebase, ch)],
                                  src_v.at[poff + b], isem).wait()
            pltpu.make_async_copy(dst_hbm.at[pl.ds(ebase, ch)],
                                  dst_v.at[poff + b], isem).wait()

    def drain_scatter(poff):
        for b in range(KB):
            pltpu.make_async_copy(rows_v.at[poff + b],
                                  acc.at[dst_v.at[poff + b]], ssem).wait()
            if accd is not None:
                pltpu.make_async_copy(ones_v,
                                      accd.at[dst_v.at[poff + b]], osem).wait()

    fire_idx(0, 0)

    def group(g, carry):
        poff = lax.rem(g, 2) * KB
        pnoff = KB - poff
        drain_idx(poff)
        gc = []
        for b in range(KB):
            gc.append(pltpu.async_copy(tbl.at[src_v.at[poff + b]],
                                       rows_v.at[poff + b], gsem[b]))

        @pl.when(g > 0)
        def _():
            drain_scatter(pnoff)

        fire_idx(jnp.minimum(g + 1, ngrp - 1), pnoff)
        for b in range(KB):
            gc[b].wait()
            pltpu.async_copy(rows_v.at[poff + b], acc.at[dst_v.at[poff + b]],
                             ssem, add=True)
            if accd is not None:
                pltpu.async_copy(ones_v, accd.at[dst_v.at[poff + b]], osem,
                                 add=True)
        return carry

    lax.fori_loop(0, ngrp, group, 0)
    last_poff = ((ngrp - 1) % 2) * KB
    drain_scatter(last_poff)
    drain_idx(KB - last_poff)


def _agg_body(h_hbm, src_hbm, dst_hbm, zrow_hbm, zdeg_hbm,
              p_out, deg_out, acc, accd, src_v, dst_v, ones_v, rows_v,
              isem, gsem, ssem, osem):
    cid = lax.axis_index("c")
    sid = lax.axis_index("s")
    wid = cid * NSUB + sid

    pltpu.sync_copy(zrow_hbm, acc.at[pl.ds(sid * RPT, RPT)])
    pltpu.sync_copy(zdeg_hbm, accd.at[pl.ds(sid * RPT, RPT)])
    ones_v[...] = jnp.ones((CH,), jnp.float32)
    plsc.subcore_barrier()

    _ring(h_hbm, src_hbm, dst_hbm, acc, src_v, dst_v, rows_v, isem, gsem,
          ssem, wid * EPW, NGRP, CH, ones_v=ones_v, accd=accd, osem=osem)
    plsc.subcore_barrier()

    pltpu.sync_copy(acc.at[pl.ds(sid * RPT, RPT)],
                    p_out.at[cid, pl.ds(sid * RPT, RPT)])
    pltpu.sync_copy(accd.at[pl.ds(sid * RPT, RPT)],
                    deg_out.at[cid, pl.ds(sid * RPT, RPT)])


def _agg128(h, src, dst, zrow, zdeg):
    return pl.kernel(
        _agg_body,
        out_type=(jax.ShapeDtypeStruct((NCORE, NP, D0), jnp.float32),
                  jax.ShapeDtypeStruct((NCORE, NP), jnp.float32)),
        mesh=_sc_mesh,
        scratch_types=[
            pltpu.VMEM_SHARED((NP, D0), jnp.float32),
            pltpu.VMEM_SHARED((NP,), jnp.float32),
            pltpu.VMEM((2 * KB, CH), jnp.int32),
            pltpu.VMEM((2 * KB, CH), jnp.int32),
            pltpu.VMEM((CH,), jnp.float32),
            pltpu.VMEM((2 * KB, CH, D0), jnp.float32),
            pltpu.SemaphoreType.DMA,
            [pltpu.SemaphoreType.DMA] * KB,
            pltpu.SemaphoreType.DMA,
            pltpu.SemaphoreType.DMA,
        ],
    )(h, src, dst, zrow, zdeg)


def _agg16_body(z_hbm, src_hbm, dst_hbm, z16_hbm, q_out,
                acc, src_v, dst_v, rows_v, isem, gsem, ssem):
    cid = lax.axis_index("c")
    sid = lax.axis_index("s")
    wid = cid * NSUB + sid

    pltpu.sync_copy(z16_hbm, acc.at[pl.ds(sid * RPT, RPT)])
    plsc.subcore_barrier()

    _ring(z_hbm, src_hbm, dst_hbm, acc, src_v, dst_v, rows_v, isem, gsem,
          ssem, wid * EPW, NGRP1, CH1)
    plsc.subcore_barrier()

    pltpu.sync_copy(acc.at[pl.ds(sid * RPT, RPT)],
                    q_out.at[cid, pl.ds(sid * RPT, RPT)])


def _agg16(z1, src, dst, z16):
    return pl.kernel(
        _agg16_body,
        out_type=jax.ShapeDtypeStruct((NCORE, NP, DP), jnp.float32),
        mesh=_sc_mesh,
        compiler_params=pltpu.CompilerParams(use_tc_tiling_on_sc=False),
        scratch_types=[
            pltpu.VMEM_SHARED((NP, DP), jnp.float32),
            pltpu.VMEM((2 * KB, CH1), jnp.int32),
            pltpu.VMEM((2 * KB, CH1), jnp.int32),
            pltpu.VMEM((2 * KB, CH1, DP), jnp.float32),
            pltpu.SemaphoreType.DMA,
            [pltpu.SemaphoreType.DMA] * KB,
            pltpu.SemaphoreType.DMA,
        ],
    )(z1, src, dst, z16)


def _mid_body(eps_ref, h_r, p0_r, p1_r, deg_r, w0_r, b0_r, g0_r, be0_r,
              w1_r, wp_r, z1_o, zp_o, dinv_o):
    deg = deg_r[0, :] + deg_r[1, :]
    dinv = 1.0 / jnp.maximum(deg, 1.0)
    neigh = (p0_r[...] + p1_r[...]) * dinv[:, None]
    t0 = (1.0 + eps_ref[0]) * h_r[...] + neigh
    a = jnp.dot(t0, w0_r[...], preferred_element_type=jnp.float32) + b0_r[...]
    h0 = jnp.maximum(a * (g0_r[...] * _BNI) + be0_r[...], 0.0)
    z1_o[...] = jnp.dot(h0, w1_r[...], preferred_element_type=jnp.float32)
    zp_o[...] = jnp.dot(h0, wp_r[...], preferred_element_type=jnp.float32)
    dinv_o[...] = jnp.broadcast_to(dinv[:, None], dinv_o.shape)


def _mid(hp, p, degp, eps0, W0, b0, g0, be0, W1p, Wpp):
    R = 1024
    grid = NP // R
    row = lambda i: (i, 0)
    full = lambda i: (0, 0)
    return pl.pallas_call(
        _mid_body,
        grid=(grid,),
        in_specs=[
            pl.BlockSpec(memory_space=pltpu.SMEM),
            pl.BlockSpec((R, D0), row),
            pl.BlockSpec((R, D0), row),
            pl.BlockSpec((R, D0), row),
            pl.BlockSpec((NCORE, R), lambda i: (0, i)),
            pl.BlockSpec((D0, D0), full),
            pl.BlockSpec((1, D0), full),
            pl.BlockSpec((1, D0), full),
            pl.BlockSpec((1, D0), full),
            pl.BlockSpec((D0, DP), full),
            pl.BlockSpec((D0, DP), full),
        ],
        out_specs=[
            pl.BlockSpec((R, DP), row),
            pl.BlockSpec((R, DP), row),
            pl.BlockSpec((R, DP), row),
        ],
        out_shape=[
            jax.ShapeDtypeStruct((NP, DP), jnp.float32),
            jax.ShapeDtypeStruct((NP, DP), jnp.float32),
            jax.ShapeDtypeStruct((NP, DP), jnp.float32),
        ],
    )(eps0, hp, p[0], p[1], degp, W0, b0, g0, be0, W1p, Wpp)


def _fin_body(eps_ref, z1_r, q0_r, q1_r, dinv_r, zp_r, b1_r, g1_r, be1_r, o_r):
    t1 = (1.0 + eps_ref[0]) * z1_r[...] + (q0_r[...] + q1_r[...]) * dinv_r[...]
    h1 = jnp.maximum((t1 + b1_r[...]) * (g1_r[...] * _BNI) + be1_r[...], 0.0)
    o_r[...] = (zp_r[...] + h1) * 0.5


def _fin(z1, q, dinv, zp, eps1, b1p, g1p, be1p):
    R = 1024
    grid = NP // R
    row = lambda i: (i, 0)
    full = lambda i: (0, 0)
    return pl.pallas_call(
        _fin_body,
        grid=(grid,),
        in_specs=[
            pl.BlockSpec(memory_space=pltpu.SMEM),
            pl.BlockSpec((R, DP), row),
            pl.BlockSpec((R, DP), row),
            pl.BlockSpec((R, DP), row),
            pl.BlockSpec((R, DP), row),
            pl.BlockSpec((R, DP), row),
            pl.BlockSpec((1, DP), full),
            pl.BlockSpec((1, DP), full),
            pl.BlockSpec((1, DP), full),
        ],
        out_specs=pl.BlockSpec((R, DP), row),
        out_shape=jax.ShapeDtypeStruct((NP, DP), jnp.float32),
    )(eps1, z1, q[0], q[1], dinv, zp, b1p, g1p, be1p)


def kernel(g, h, snorm_n, snorm_e, eps0, W0, b0, bn0_g, bn0_b,
           eps1, W1, b1, bn1_g, bn1_b, Wp):
    del snorm_n, snorm_e
    f32 = jnp.float32
    npad = EP - E
    fill = jnp.arange(npad, dtype=jnp.int32)
    src = jnp.concatenate([g[0], fill % 997])
    dst = jnp.concatenate([g[1], N + fill % (NP - N)])
    hp = jnp.pad(h, ((0, NP - N), (0, 0)))
    zrow = jnp.zeros((RPT, D0), f32)
    zdeg = jnp.zeros((RPT,), f32)
    z16 = jnp.zeros((RPT, DP), f32)
    W1p = jnp.pad(W1, ((0, 0), (0, DP - W1.shape[1])))
    Wpp = jnp.pad(Wp, ((0, 0), (0, DP - Wp.shape[1])))
    b1p = jnp.pad(b1, (0, DP - b1.shape[0])).reshape(1, DP)
    g1p = jnp.pad(bn1_g, (0, DP - bn1_g.shape[0])).reshape(1, DP)
    be1p = jnp.pad(bn1_b, (0, DP - bn1_b.shape[0])).reshape(1, DP)

    p, degp = _agg128(h, src, dst, zrow, zdeg)
    z1, zp, dinv = _mid(hp, p, degp, eps0, W0, b0.reshape(1, D0),
                        bn0_g.reshape(1, D0), bn0_b.reshape(1, D0), W1p, Wpp)
    q = _agg16(z1, src, dst, z16)
    out = _fin(z1, q, dinv, zp, eps1, b1p, g1p, be1p)
    return out[:N, :Wp.shape[1]]
